# baseline (speedup 1.0000x reference)
import functools

import jax
import jax.numpy as jnp
from jax import lax
from jax.experimental import pallas as pl
from jax.experimental.pallas import tpu as pltpu
from jax.experimental.pallas import tpu_sc as plsc

N = 10000
E = 320000
D = 128

CH = 128
NCHUNK = E // CH
NW = 32
NSLOT = 2
NT = NCHUNK // (NW * NSLOT)
NPAIR = NT // 2
NREM = NT - 2 * NPAIR
NTAIL = NCHUNK - NT * NW * NSLOT
NGROUP = N // 8
G_BASE = NGROUP // 16
G_EXTRA = NGROUP % 16
DRAIN = 104


def _linear_body(x_ref, w_ref, b_ref, o_ref):
    o_ref[...] = lax.dot_general(
        x_ref[...], w_ref[...], (((1,), (1,)), ((), ())),
        preferred_element_type=jnp.float32) + b_ref[...]


def _linear(x, W, b):
    blk = 1000
    return pl.pallas_call(
        _linear_body,
        grid=(N // blk,),
        in_specs=[
            pl.BlockSpec((blk, D), lambda i: (i, 0)),
            pl.BlockSpec((D, D), lambda i: (0, 0)),
            pl.BlockSpec((1, D), lambda i: (0, 0)),
        ],
        out_specs=pl.BlockSpec((blk, D), lambda i: (i, 0)),
        out_shape=jax.ShapeDtypeStruct((N, D), jnp.float32),
    )(x, W, b.reshape(1, D))


_SPLAT_DNUMS = lax.GatherDimensionNumbers(
    offset_dims=(), collapsed_slice_dims=(0,), start_index_map=(0,))


def _splat(vec16, lane):
    idx = jnp.full((16, 1), lane, jnp.int32)
    return lax.gather(vec16, idx, _SPLAT_DNUMS, slice_sizes=(1,),
                      mode=lax.GatherScatterMode.PROMISE_IN_BOUNDS)


def _spmm_body(h_hbm, dst_hbm, src_hbm, ew_hbm, out_hbm,
               dstn, srcn, ewn, rows_s, acc_sh, esems, gsems, ssems):
    cid = lax.axis_index("c")
    sid = lax.axis_index("s")
    wid = sid * 2 + cid
    row0 = pl.multiple_of(8 * (G_BASE * sid + jnp.minimum(sid, G_EXTRA)), 8)
    has_extra = sid < G_EXTRA

    def _edata_descs(chunk, s, p):
        base = pl.multiple_of(chunk * CH, CH)
        return (
            pltpu.make_async_copy(dst_hbm.at[pl.ds(base, CH)],
                                  dstn.at[s, p], esems[s]),
            pltpu.make_async_copy(src_hbm.at[pl.ds(base, CH)],
                                  srcn.at[s, p], esems[s]),
            pltpu.make_async_copy(ew_hbm.at[pl.ds(base, CH)],
                                  ewn.at[s, p], esems[s]),
        )

    def _gather_desc(s, p):
        return pltpu.make_async_copy(h_hbm.at[srcn.at[s, p]], rows_s[s],
                                     gsems[s])

    def _scatter_desc(s, p):
        return pltpu.make_async_copy(rows_s[s], acc_sh.at[dstn.at[s, p]],
                                     ssems[s])

    def _scale_slot(s, p):
        @plsc.parallel_loop(0, CH // 16, unroll=2)
        def _scale(g):
            ew16 = ewn[s, p, pl.ds(pl.multiple_of(g * 16, 16), 16)]
            for lane in range(16):
                e = g * 16 + lane
                wv = _splat(ew16, lane)
                for j in range(D // 16):
                    sl = pl.ds(j * 16, 16)
                    rows_s[s][e, sl] = rows_s[s][e, sl] * wv

    zeros16 = jnp.zeros((16,), jnp.float32)

    def _zrow(i, carry):
        for j in range(D // 16):
            rows_s[0][i, pl.ds(j * 16, 16)] = zeros16
        return carry

    lax.fori_loop(0, DRAIN, _zrow, 0)
    for r in range(G_BASE * 8 // DRAIN):
        pltpu.sync_copy(rows_s[0].at[pl.ds(0, DRAIN)],
                        acc_sh.at[pl.ds(pl.multiple_of(row0 + r * DRAIN, 8),
                                        DRAIN)])

    @pl.when(has_extra)
    def _():
        pltpu.sync_copy(
            rows_s[0].at[pl.ds(0, 8)],
            acc_sh.at[pl.ds(pl.multiple_of(row0 + G_BASE * 8, 8), 8)])

    plsc.subcore_barrier()

    def _chunk_of(t, s):
        return (t * NSLOT + s) * NW + wid

    for s in range(NSLOT):
        for d in _edata_descs(_chunk_of(0, s), s, 0):
            d.start()
    for s in range(NSLOT):
        for d in _edata_descs(_chunk_of(0, s), s, 0):
            d.wait()
        _gather_desc(s, 0).start()
        for d in _edata_descs(_chunk_of(1, s), s, 1):
            d.start()

    def _phase_a(t, p):
        for s in range(NSLOT):
            _gather_desc(s, p).wait()
            _scale_slot(s, p)
            _scatter_desc(s, p).start(add=True)

    def _phase_b(t, p, traced=True):
        for s in range(NSLOT):
            _scatter_desc(s, p).wait()
            if traced or t + 1 < NT:
                @pl.when(t + 1 < NT)
                def _(s=s):
                    for d in _edata_descs(_chunk_of(t + 1, s), s, 1 - p):
                        d.wait()
                    _gather_desc(s, 1 - p).start()
            if traced or t + 2 < NT:
                @pl.when(t + 2 < NT)
                def _(s=s):
                    for d in _edata_descs(_chunk_of(t + 2, s), s, p):
                        d.start()

    def _pair(th, carry):
        t0 = th * 2
        _phase_a(t0, 0)
        _phase_b(t0, 0)
        _phase_a(t0 + 1, 1)
        _phase_b(t0 + 1, 1)
        return carry

    lax.fori_loop(0, NPAIR, _pair, 0)

    for _ in range(NREM):
        _phase_a(NT - 1, 0)
        for s in range(NSLOT):
            _scatter_desc(s, 0).wait()

    @pl.when(wid < NTAIL)
    def _():
        for d in _edata_descs(NT * NSLOT * NW + wid, 0, 0):
            d.start()
        for d in _edata_descs(NT * NSLOT * NW + wid, 0, 0):
            d.wait()
        _gather_desc(0, 0).start()
        _gather_desc(0, 0).wait()
        _scale_slot(0, 0)
        _scatter_desc(0, 0).start(add=True)
        _scatter_desc(0, 0).wait()

    plsc.subcore_barrier()

    for r in range(G_BASE * 8 // DRAIN):
        sl = pl.ds(pl.multiple_of(row0 + r * DRAIN, 8), DRAIN)
        pltpu.sync_copy(acc_sh.at[sl], rows_s[0].at[pl.ds(0, DRAIN)])
        pltpu.sync_copy(rows_s[0].at[pl.ds(0, DRAIN)], out_hbm.at[cid, sl])

    @pl.when(has_extra)
    def _():
        sl = pl.ds(pl.multiple_of(row0 + G_BASE * 8, 8), 8)
        pltpu.sync_copy(acc_sh.at[sl], rows_s[0].at[pl.ds(0, 8)])
        pltpu.sync_copy(rows_s[0].at[pl.ds(0, 8)], out_hbm.at[cid, sl])


def _spmm_sc(h, dst, src, ew):
    mesh = plsc.VectorSubcoreMesh(core_axis_name="c", subcore_axis_name="s")
    f = functools.partial(
        pl.kernel,
        out_type=jax.ShapeDtypeStruct((2, N, D), jnp.float32),
        mesh=mesh,
        scratch_types=[
            pltpu.VMEM((NSLOT, 2, CH), jnp.int32),
            pltpu.VMEM((NSLOT, 2, CH), jnp.int32),
            pltpu.VMEM((NSLOT, 2, CH), jnp.float32),
            [pltpu.VMEM((CH, D), jnp.float32) for _ in range(NSLOT)],
            pltpu.VMEM_SHARED((N, D), jnp.float32),
            [pltpu.SemaphoreType.DMA for _ in range(NSLOT)],
            [pltpu.SemaphoreType.DMA for _ in range(NSLOT)],
            [pltpu.SemaphoreType.DMA for _ in range(NSLOT)],
        ],
    )(_spmm_body)
    return f(h, dst, src, ew)


def _comb_body(p_ref, o_ref):
    o_ref[...] = p_ref[0] + p_ref[1]


def _combine(parts):
    blk = 1000
    return pl.pallas_call(
        _comb_body,
        grid=(N // blk,),
        in_specs=[pl.BlockSpec((2, blk, D), lambda i: (0, i, 0))],
        out_specs=pl.BlockSpec((blk, D), lambda i: (i, 0)),
        out_shape=jax.ShapeDtypeStruct((N, D), jnp.float32),
    )(parts)


def kernel(x, edge_index, edge_weight, W, b):
    h = _linear(x, W, b)
    parts = _spmm_sc(h, edge_index[0], edge_index[1], edge_weight)
    return _combine(parts)


# speedup vs baseline: 1.0650x; 1.0650x over previous
import functools

import jax
import jax.numpy as jnp
from jax import lax
from jax.experimental import pallas as pl
from jax.experimental.pallas import tpu as pltpu
from jax.experimental.pallas import tpu_sc as plsc

N = 10000
E = 320000
D = 128

CH = 64
NCHUNK = E // CH
NW = 32
NSLOT = 2
NT = NCHUNK // (NW * NSLOT)
NTRI = NT // 3
NTAIL = NCHUNK - NT * NW * NSLOT
NGROUP = N // 8
G_BASE = NGROUP // 16
G_EXTRA = NGROUP % 16
DRAIN = 48

assert NT % 3 == 0


def _linear_body(x_ref, w_ref, b_ref, o_ref):
    o_ref[...] = lax.dot_general(
        x_ref[...], w_ref[...], (((1,), (1,)), ((), ())),
        preferred_element_type=jnp.float32) + b_ref[...]


def _linear(x, W, b):
    blk = 1000
    return pl.pallas_call(
        _linear_body,
        grid=(N // blk,),
        in_specs=[
            pl.BlockSpec((blk, D), lambda i: (i, 0)),
            pl.BlockSpec((D, D), lambda i: (0, 0)),
            pl.BlockSpec((1, D), lambda i: (0, 0)),
        ],
        out_specs=pl.BlockSpec((blk, D), lambda i: (i, 0)),
        out_shape=jax.ShapeDtypeStruct((N, D), jnp.float32),
    )(x, W, b.reshape(1, D))


_SPLAT_DNUMS = lax.GatherDimensionNumbers(
    offset_dims=(), collapsed_slice_dims=(0,), start_index_map=(0,))


def _splat(vec16, lane):
    idx = jnp.full((16, 1), lane, jnp.int32)
    return lax.gather(vec16, idx, _SPLAT_DNUMS, slice_sizes=(1,),
                      mode=lax.GatherScatterMode.PROMISE_IN_BOUNDS)


def _spmm_body(h_hbm, dst_hbm, src_hbm, ew_hbm, out_hbm,
               dstn, srcn, ewn, rows_s, scat_s, acc_sh,
               esems, gsems, ssems):
    cid = lax.axis_index("c")
    sid = lax.axis_index("s")
    wid = sid * 2 + cid
    row0 = pl.multiple_of(8 * (G_BASE * sid + jnp.minimum(sid, G_EXTRA)), 8)
    has_extra = sid < G_EXTRA

    def _edata_descs(chunk, s, q):
        base = pl.multiple_of(chunk * CH, CH)
        return (
            pltpu.make_async_copy(dst_hbm.at[pl.ds(base, CH)],
                                  dstn.at[s, q], esems[s]),
            pltpu.make_async_copy(src_hbm.at[pl.ds(base, CH)],
                                  srcn.at[s, q], esems[s]),
            pltpu.make_async_copy(ew_hbm.at[pl.ds(base, CH)],
                                  ewn.at[s, q], esems[s]),
        )

    def _gather_desc(s, q):
        return pltpu.make_async_copy(h_hbm.at[srcn.at[s, q]], rows_s[s],
                                     gsems[s])

    def _scatter_desc(s, q):
        return pltpu.make_async_copy(scat_s[s], acc_sh.at[dstn.at[s, q]],
                                     ssems[s])

    def _scale_slot(s, q):
        @plsc.parallel_loop(0, CH // 16, unroll=2)
        def _scale(g):
            ew16 = ewn[s, q, pl.ds(pl.multiple_of(g * 16, 16), 16)]
            for lane in range(16):
                e = g * 16 + lane
                wv = _splat(ew16, lane)
                for j in range(D // 16):
                    sl = pl.ds(j * 16, 16)
                    scat_s[s][e, sl] = rows_s[s][e, sl] * wv

    zeros16 = jnp.zeros((16,), jnp.float32)

    def _zrow(i, carry):
        for j in range(D // 16):
            scat_s[0][i, pl.ds(j * 16, 16)] = zeros16
        return carry

    lax.fori_loop(0, DRAIN, _zrow, 0)
    for r in range(G_BASE * 8 // DRAIN):
        pltpu.sync_copy(scat_s[0].at[pl.ds(0, DRAIN)],
                        acc_sh.at[pl.ds(pl.multiple_of(row0 + r * DRAIN, 8),
                                        DRAIN)])

    @pl.when(has_extra)
    def _():
        pltpu.sync_copy(
            scat_s[0].at[pl.ds(0, 8)],
            acc_sh.at[pl.ds(pl.multiple_of(row0 + G_BASE * 8, 8), 8)])

    plsc.subcore_barrier()

    def _chunk_of(t, s):
        return (t * NSLOT + s) * NW + wid

    for s in range(NSLOT):
        for d in _edata_descs(_chunk_of(0, s), s, 0):
            d.start()
        for d in _edata_descs(_chunk_of(1, s), s, 1):
            d.start()
    for s in range(NSLOT):
        for d in _edata_descs(_chunk_of(0, s), s, 0):
            d.wait()
        _gather_desc(s, 0).start()

    def _phase(t, q):
        q1 = (q + 1) % 3
        q2 = (q + 2) % 3
        for s in range(NSLOT):
            @pl.when(t > 0)
            def _(s=s):
                _scatter_desc(s, q2).wait()

            @pl.when(t + 2 < NT)
            def _(s=s):
                for d in _edata_descs(_chunk_of(t + 2, s), s, q2):
                    d.start()

            _gather_desc(s, q).wait()
            _scale_slot(s, q)
            _scatter_desc(s, q).start(add=True)

            @pl.when(t + 1 < NT)
            def _(s=s):
                for d in _edata_descs(_chunk_of(t + 1, s), s, q1):
                    d.wait()
                _gather_desc(s, q1).start()

    def _triple(tr, carry):
        t0 = tr * 3
        _phase(t0, 0)
        _phase(t0 + 1, 1)
        _phase(t0 + 2, 2)
        return carry

    lax.fori_loop(0, NTRI, _triple, 0)

    for s in range(NSLOT):
        _scatter_desc(s, 0).wait()

    @pl.when(wid < NTAIL)
    def _():
        for d in _edata_descs(NT * NSLOT * NW + wid, 0, 0):
            d.start()
        for d in _edata_descs(NT * NSLOT * NW + wid, 0, 0):
            d.wait()
        _gather_desc(0, 0).start()
        _gather_desc(0, 0).wait()
        _scale_slot(0, 0)
        _scatter_desc(0, 0).start(add=True)
        _scatter_desc(0, 0).wait()

    plsc.subcore_barrier()

    for r in range(G_BASE * 8 // DRAIN):
        sl = pl.ds(pl.multiple_of(row0 + r * DRAIN, 8), DRAIN)
        pltpu.sync_copy(acc_sh.at[sl], scat_s[0].at[pl.ds(0, DRAIN)])
        pltpu.sync_copy(scat_s[0].at[pl.ds(0, DRAIN)], out_hbm.at[cid, sl])

    @pl.when(has_extra)
    def _():
        sl = pl.ds(pl.multiple_of(row0 + G_BASE * 8, 8), 8)
        pltpu.sync_copy(acc_sh.at[sl], scat_s[0].at[pl.ds(0, 8)])
        pltpu.sync_copy(scat_s[0].at[pl.ds(0, 8)], out_hbm.at[cid, sl])


def _spmm_sc(h, dst, src, ew):
    mesh = plsc.VectorSubcoreMesh(core_axis_name="c", subcore_axis_name="s")
    f = functools.partial(
        pl.kernel,
        out_type=jax.ShapeDtypeStruct((2, N, D), jnp.float32),
        mesh=mesh,
        scratch_types=[
            pltpu.VMEM((NSLOT, 3, CH), jnp.int32),
            pltpu.VMEM((NSLOT, 3, CH), jnp.int32),
            pltpu.VMEM((NSLOT, 3, CH), jnp.float32),
            [pltpu.VMEM((CH, D), jnp.float32) for _ in range(NSLOT)],
            [pltpu.VMEM((CH, D), jnp.float32) for _ in range(NSLOT)],
            pltpu.VMEM_SHARED((N, D), jnp.float32),
            [pltpu.SemaphoreType.DMA for _ in range(NSLOT)],
            [pltpu.SemaphoreType.DMA for _ in range(NSLOT)],
            [pltpu.SemaphoreType.DMA for _ in range(NSLOT)],
        ],
    )(_spmm_body)
    return f(h, dst, src, ew)


def _comb_body(p_ref, o_ref):
    o_ref[...] = p_ref[0] + p_ref[1]


def _combine(parts):
    blk = 1000
    return pl.pallas_call(
        _comb_body,
        grid=(N // blk,),
        in_specs=[pl.BlockSpec((2, blk, D), lambda i: (0, i, 0))],
        out_specs=pl.BlockSpec((blk, D), lambda i: (i, 0)),
        out_shape=jax.ShapeDtypeStruct((N, D), jnp.float32),
    )(parts)


def kernel(x, edge_index, edge_weight, W, b):
    h = _linear(x, W, b)
    parts = _spmm_sc(h, edge_index[0], edge_index[1], edge_weight)
    return _combine(parts)


# speedup vs baseline: 1.0671x; 1.0020x over previous
import functools

import jax
import jax.numpy as jnp
from jax import lax
from jax.experimental import pallas as pl
from jax.experimental.pallas import tpu as pltpu
from jax.experimental.pallas import tpu_sc as plsc

N = 10000
E = 320000
D = 128

CH = 64
NCHUNK = E // CH
NW = 32
NSLOT = 2
NT = NCHUNK // (NW * NSLOT)
NTRI = NT // 3
NTAIL = NCHUNK - NT * NW * NSLOT
NGROUP = N // 8
G_BASE = NGROUP // 16
G_EXTRA = NGROUP % 16
DRAIN = 48

assert NT % 3 == 0


def _linear_body(x_ref, w_ref, b_ref, o_ref):
    o_ref[...] = lax.dot_general(
        x_ref[...], w_ref[...], (((1,), (1,)), ((), ())),
        preferred_element_type=jnp.float32) + b_ref[...]


def _linear(x, W, b):
    blk = 1000
    return pl.pallas_call(
        _linear_body,
        grid=(N // blk,),
        in_specs=[
            pl.BlockSpec((blk, D), lambda i: (i, 0)),
            pl.BlockSpec((D, D), lambda i: (0, 0)),
            pl.BlockSpec((1, D), lambda i: (0, 0)),
        ],
        out_specs=pl.BlockSpec((blk, D), lambda i: (i, 0)),
        out_shape=jax.ShapeDtypeStruct((N, D), jnp.float32),
    )(x, W, b.reshape(1, D))


_SPLAT_DNUMS = lax.GatherDimensionNumbers(
    offset_dims=(), collapsed_slice_dims=(0,), start_index_map=(0,))


def _splat(vec16, lane):
    idx = jnp.full((16, 1), lane, jnp.int32)
    return lax.gather(vec16, idx, _SPLAT_DNUMS, slice_sizes=(1,),
                      mode=lax.GatherScatterMode.PROMISE_IN_BOUNDS)


def _spmm_body(h_hbm, dst_hbm, src_hbm, ew_hbm, out_hbm,
               dstn, srcn, ewn, rows_s, scat_s, acc_sh,
               esems, gsems, ssems):
    cid = lax.axis_index("c")
    sid = lax.axis_index("s")
    wid = sid * 2 + cid
    row0 = pl.multiple_of(8 * (G_BASE * sid + jnp.minimum(sid, G_EXTRA)), 8)
    has_extra = sid < G_EXTRA

    def _edata_descs(chunk, s, q):
        base = pl.multiple_of(chunk * CH, CH)
        return (
            pltpu.make_async_copy(dst_hbm.at[pl.ds(base, CH)],
                                  dstn.at[s, q], esems[s]),
            pltpu.make_async_copy(src_hbm.at[pl.ds(base, CH)],
                                  srcn.at[s, q], esems[s]),
            pltpu.make_async_copy(ew_hbm.at[pl.ds(base, CH)],
                                  ewn.at[s, q], esems[s]),
        )

    def _gather_desc(s, q):
        return pltpu.make_async_copy(h_hbm.at[srcn.at[s, q]], rows_s[s],
                                     gsems[s])

    def _scatter_desc(s, q):
        return pltpu.make_async_copy(scat_s[s], acc_sh.at[dstn.at[s, q]],
                                     ssems[s])

    def _scale_slot(s, q):
        @plsc.parallel_loop(0, CH // 16, unroll=2)
        def _scale(g):
            ew16 = ewn[s, q, pl.ds(pl.multiple_of(g * 16, 16), 16)]
            for lane in range(16):
                e = g * 16 + lane
                wv = _splat(ew16, lane)
                for j in range(D // 16):
                    sl = pl.ds(j * 16, 16)
                    scat_s[s][e, sl] = rows_s[s][e, sl] * wv

    def _chunk_of(t, s):
        return (t * NSLOT + s) * NW + wid

    for s in range(NSLOT):
        for d in _edata_descs(_chunk_of(0, s), s, 0):
            d.start()
        for d in _edata_descs(_chunk_of(1, s), s, 1):
            d.start()
    for s in range(NSLOT):
        for d in _edata_descs(_chunk_of(0, s), s, 0):
            d.wait()
        _gather_desc(s, 0).start()

    zeros16 = jnp.zeros((16,), jnp.float32)

    def _zrow(i, carry):
        for j in range(D // 16):
            scat_s[0][i, pl.ds(j * 16, 16)] = zeros16
        return carry

    lax.fori_loop(0, DRAIN, _zrow, 0)
    for r in range(G_BASE * 8 // DRAIN):
        pltpu.sync_copy(scat_s[0].at[pl.ds(0, DRAIN)],
                        acc_sh.at[pl.ds(pl.multiple_of(row0 + r * DRAIN, 8),
                                        DRAIN)])

    @pl.when(has_extra)
    def _():
        pltpu.sync_copy(
            scat_s[0].at[pl.ds(0, 8)],
            acc_sh.at[pl.ds(pl.multiple_of(row0 + G_BASE * 8, 8), 8)])

    plsc.subcore_barrier()

    def _phase(t, q):
        q1 = (q + 1) % 3
        q2 = (q + 2) % 3
        for s in range(NSLOT):
            @pl.when(t > 0)
            def _(s=s):
                _scatter_desc(s, q2).wait()

            @pl.when(t + 2 < NT)
            def _(s=s):
                for d in _edata_descs(_chunk_of(t + 2, s), s, q2):
                    d.start()

            _gather_desc(s, q).wait()
            _scale_slot(s, q)
            _scatter_desc(s, q).start(add=True)

            @pl.when(t + 1 < NT)
            def _(s=s):
                for d in _edata_descs(_chunk_of(t + 1, s), s, q1):
                    d.wait()
                _gather_desc(s, q1).start()

    def _triple(tr, carry):
        t0 = tr * 3
        _phase(t0, 0)
        _phase(t0 + 1, 1)
        _phase(t0 + 2, 2)
        return carry

    lax.fori_loop(0, NTRI, _triple, 0)

    for s in range(NSLOT):
        _scatter_desc(s, 0).wait()

    @pl.when(wid < NTAIL)
    def _():
        for d in _edata_descs(NT * NSLOT * NW + wid, 0, 0):
            d.start()
        for d in _edata_descs(NT * NSLOT * NW + wid, 0, 0):
            d.wait()
        _gather_desc(0, 0).start()
        _gather_desc(0, 0).wait()
        _scale_slot(0, 0)
        _scatter_desc(0, 0).start(add=True)
        _scatter_desc(0, 0).wait()

    plsc.subcore_barrier()

    for r in range(G_BASE * 8 // DRAIN):
        sl = pl.ds(pl.multiple_of(row0 + r * DRAIN, 8), DRAIN)
        pltpu.sync_copy(acc_sh.at[sl], out_hbm.at[cid, sl])

    @pl.when(has_extra)
    def _():
        sl = pl.ds(pl.multiple_of(row0 + G_BASE * 8, 8), 8)
        pltpu.sync_copy(acc_sh.at[sl], out_hbm.at[cid, sl])


def _spmm_sc(h, dst, src, ew):
    mesh = plsc.VectorSubcoreMesh(core_axis_name="c", subcore_axis_name="s")
    f = functools.partial(
        pl.kernel,
        out_type=jax.ShapeDtypeStruct((2, N, D), jnp.float32),
        mesh=mesh,
        scratch_types=[
            pltpu.VMEM((NSLOT, 3, CH), jnp.int32),
            pltpu.VMEM((NSLOT, 3, CH), jnp.int32),
            pltpu.VMEM((NSLOT, 3, CH), jnp.float32),
            [pltpu.VMEM((CH, D), jnp.float32) for _ in range(NSLOT)],
            [pltpu.VMEM((CH, D), jnp.float32) for _ in range(NSLOT)],
            pltpu.VMEM_SHARED((N, D), jnp.float32),
            [pltpu.SemaphoreType.DMA for _ in range(NSLOT)],
            [pltpu.SemaphoreType.DMA for _ in range(NSLOT)],
            [pltpu.SemaphoreType.DMA for _ in range(NSLOT)],
        ],
    )(_spmm_body)
    return f(h, dst, src, ew)


def _comb_body(p_ref, o_ref):
    o_ref[...] = p_ref[0] + p_ref[1]


def _combine(parts):
    blk = 1000
    return pl.pallas_call(
        _comb_body,
        grid=(N // blk,),
        in_specs=[pl.BlockSpec((2, blk, D), lambda i: (0, i, 0))],
        out_specs=pl.BlockSpec((blk, D), lambda i: (i, 0)),
        out_shape=jax.ShapeDtypeStruct((N, D), jnp.float32),
    )(parts)


def kernel(x, edge_index, edge_weight, W, b):
    h = _linear(x, W, b)
    parts = _spmm_sc(h, edge_index[0], edge_index[1], edge_weight)
    return _combine(parts)


# speedup vs baseline: 1.0802x; 1.0123x over previous
import functools

import jax
import jax.numpy as jnp
from jax import lax
from jax.experimental import pallas as pl
from jax.experimental.pallas import tpu as pltpu
from jax.experimental.pallas import tpu_sc as plsc

N = 10000
E = 320000
D = 128

CH = 64
NCHUNK = E // CH
NW = 32
NSLOT = 4
NQUAD = NCHUNK // NSLOT
Q_EXTRA = NQUAD % NW
NGROUP = N // 8
G_BASE = NGROUP // 16
G_EXTRA = NGROUP % 16
DRAIN = 48


def _linear_body(x_ref, w_ref, b_ref, o_ref):
    o_ref[...] = lax.dot_general(
        x_ref[...], w_ref[...], (((1,), (1,)), ((), ())),
        preferred_element_type=jnp.float32) + b_ref[...]


def _linear(x, W, b):
    blk = 1000
    return pl.pallas_call(
        _linear_body,
        grid=(N // blk,),
        in_specs=[
            pl.BlockSpec((blk, D), lambda i: (i, 0)),
            pl.BlockSpec((D, D), lambda i: (0, 0)),
            pl.BlockSpec((1, D), lambda i: (0, 0)),
        ],
        out_specs=pl.BlockSpec((blk, D), lambda i: (i, 0)),
        out_shape=jax.ShapeDtypeStruct((N, D), jnp.float32),
    )(x, W, b.reshape(1, D))


_SPLAT_DNUMS = lax.GatherDimensionNumbers(
    offset_dims=(), collapsed_slice_dims=(0,), start_index_map=(0,))


def _splat(vec16, lane):
    idx = jnp.full((16, 1), lane, jnp.int32)
    return lax.gather(vec16, idx, _SPLAT_DNUMS, slice_sizes=(1,),
                      mode=lax.GatherScatterMode.PROMISE_IN_BOUNDS)


def _spmm_body(h_hbm, dst_hbm, src_hbm, ew_hbm, out_hbm,
               dst_s, src_s, ew_s, rows_s, didx, acc_sh,
               esems, gsems, ssems):
    cid = lax.axis_index("c")
    sid = lax.axis_index("s")
    wid = sid * 2 + cid
    row0 = pl.multiple_of(8 * (G_BASE * sid + jnp.minimum(sid, G_EXTRA)), 8)
    has_extra = sid < G_EXTRA

    def _edata_descs(quad, s):
        base = pl.multiple_of((quad * NSLOT + s) * CH, CH)
        return (
            pltpu.make_async_copy(dst_hbm.at[pl.ds(base, CH)], dst_s[s],
                                  esems[s]),
            pltpu.make_async_copy(src_hbm.at[pl.ds(base, CH)], src_s[s],
                                  esems[s]),
            pltpu.make_async_copy(ew_hbm.at[pl.ds(base, CH)], ew_s[s],
                                  esems[s]),
        )

    def _gather_desc(s):
        return pltpu.make_async_copy(h_hbm.at[src_s[s]], rows_s[s], gsems[s])

    def _scatter_desc(s):
        return pltpu.make_async_copy(rows_s[s], acc_sh.at[didx.at[s]],
                                     ssems[s])

    def _scale_slot(s):
        @plsc.parallel_loop(0, CH // 16, unroll=2)
        def _scale(g):
            ew16 = ew_s[s][pl.ds(pl.multiple_of(g * 16, 16), 16)]
            for lane in range(16):
                e = g * 16 + lane
                wv = _splat(ew16, lane)
                for j in range(D // 16):
                    sl = pl.ds(j * 16, 16)
                    rows_s[s][e, sl] = rows_s[s][e, sl] * wv

    nq = (NQUAD // NW) + jnp.where(wid < Q_EXTRA, 1, 0)

    @pl.when(nq > 0)
    def _():
        for s in range(NSLOT):
            for d in _edata_descs(wid, s):
                d.start()

    zeros16 = jnp.zeros((16,), jnp.float32)

    def _zrow(i, carry):
        for j in range(D // 16):
            rows_s[NSLOT - 1][i, pl.ds(j * 16, 16)] = zeros16
        return carry

    lax.fori_loop(0, DRAIN, _zrow, 0)
    for r in range(G_BASE * 8 // DRAIN):
        pltpu.sync_copy(rows_s[NSLOT - 1].at[pl.ds(0, DRAIN)],
                        acc_sh.at[pl.ds(pl.multiple_of(row0 + r * DRAIN, 8),
                                        DRAIN)])

    @pl.when(has_extra)
    def _():
        pltpu.sync_copy(
            rows_s[NSLOT - 1].at[pl.ds(0, 8)],
            acc_sh.at[pl.ds(pl.multiple_of(row0 + G_BASE * 8, 8), 8)])

    plsc.subcore_barrier()

    def _quad(t, carry):
        quad = t * NW + wid

        for s in range(NSLOT):
            @pl.when(t > 0)
            def _(s=s):
                _scatter_desc(s).wait()
            for d in _edata_descs(quad, s):
                d.wait()
            for j in range(CH // 16):
                sl = pl.ds(j * 16, 16)
                didx[s, sl] = dst_s[s][sl]
            _gather_desc(s).start()

        for s in range(NSLOT):
            _gather_desc(s).wait()

            @pl.when(t + 1 < nq)
            def _(s=s):
                for d in _edata_descs(quad + NW, s):
                    d.start()

            _scale_slot(s)
            _scatter_desc(s).start(add=True)
        return carry

    lax.fori_loop(0, nq, _quad, 0)

    @pl.when(nq > 0)
    def _():
        for s in range(NSLOT):
            _scatter_desc(s).wait()

    plsc.subcore_barrier()

    for r in range(G_BASE * 8 // DRAIN):
        sl = pl.ds(pl.multiple_of(row0 + r * DRAIN, 8), DRAIN)
        pltpu.sync_copy(acc_sh.at[sl], out_hbm.at[cid, sl])

    @pl.when(has_extra)
    def _():
        sl = pl.ds(pl.multiple_of(row0 + G_BASE * 8, 8), 8)
        pltpu.sync_copy(acc_sh.at[sl], out_hbm.at[cid, sl])


def _spmm_sc(h, dst, src, ew):
    mesh = plsc.VectorSubcoreMesh(core_axis_name="c", subcore_axis_name="s")
    f = functools.partial(
        pl.kernel,
        out_type=jax.ShapeDtypeStruct((2, N, D), jnp.float32),
        mesh=mesh,
        scratch_types=[
            [pltpu.VMEM((CH,), jnp.int32) for _ in range(NSLOT)],
            [pltpu.VMEM((CH,), jnp.int32) for _ in range(NSLOT)],
            [pltpu.VMEM((CH,), jnp.float32) for _ in range(NSLOT)],
            [pltpu.VMEM((CH, D), jnp.float32) for _ in range(NSLOT)],
            pltpu.VMEM((NSLOT, CH), jnp.int32),
            pltpu.VMEM_SHARED((N, D), jnp.float32),
            [pltpu.SemaphoreType.DMA for _ in range(NSLOT)],
            [pltpu.SemaphoreType.DMA for _ in range(NSLOT)],
            [pltpu.SemaphoreType.DMA for _ in range(NSLOT)],
        ],
    )(_spmm_body)
    return f(h, dst, src, ew)


def _comb_body(p_ref, o_ref):
    o_ref[...] = p_ref[0] + p_ref[1]


def _combine(parts):
    blk = 1000
    return pl.pallas_call(
        _comb_body,
        grid=(N // blk,),
        in_specs=[pl.BlockSpec((2, blk, D), lambda i: (0, i, 0))],
        out_specs=pl.BlockSpec((blk, D), lambda i: (i, 0)),
        out_shape=jax.ShapeDtypeStruct((N, D), jnp.float32),
    )(parts)


def kernel(x, edge_index, edge_weight, W, b):
    h = _linear(x, W, b)
    parts = _spmm_sc(h, edge_index[0], edge_index[1], edge_weight)
    return _combine(parts)
